# baseline (speedup 1.0000x reference)
import jax
import jax.numpy as jnp
from jax import lax
from jax.experimental import pallas as pl
from jax.experimental.pallas import tpu as pltpu
from jax.experimental.pallas import tpu_sc as plsc

_N = 24
_E = 128


def _edge_hist_scs_body(edge_hbm, zeros_hbm, out_hbm, edge_s, cnt_s,
                        sem1, sem2):
    cp1 = pltpu.async_copy(edge_hbm, edge_s, sem1)
    cp2 = pltpu.async_copy(zeros_hbm, cnt_s, sem2)
    cp1.wait()
    cp2.wait()

    def acc(e, _):
        s0 = edge_s[0, 2 * e]
        d0 = edge_s[1, 2 * e]
        cnt_s[0, d0, s0] = cnt_s[0, d0, s0] + 1
        s1 = edge_s[0, 2 * e + 1]
        d1 = edge_s[1, 2 * e + 1]
        cnt_s[1, d1, s1] = cnt_s[1, d1, s1] + 1
        return 0

    lax.fori_loop(0, _E // 2, acc, 0)
    pltpu.sync_copy(cnt_s, out_hbm)


def _edge_hist_scs(edge_index, zeros):
    mesh = plsc.ScalarSubcoreMesh(axis_name="c", num_cores=1)
    return pl.kernel(
        _edge_hist_scs_body,
        out_type=jax.ShapeDtypeStruct((2, _N, _N), jnp.int32),
        mesh=mesh,
        compiler_params=pltpu.CompilerParams(
            needs_layout_passes=False,
            skip_device_barrier=True,
            disable_bounds_checks=True,
            disable_semaphore_checks=True,
        ),
        scratch_types=[
            pltpu.SMEM((2, _E), jnp.int32),
            pltpu.SMEM((2, _N, _N), jnp.int32),
            pltpu.SemaphoreType.DMA,
            pltpu.SemaphoreType.DMA,
        ],
    )(edge_index, zeros)


def _edge_hist_body(edge_hbm, out_hbm, edge_v, cnt_v, sem):
    cp = pltpu.async_copy(edge_hbm, edge_v, sem)
    zeros = jnp.zeros((16,), jnp.float32)
    for r in range(_N):
        cnt_v[r, pl.ds(0, 16)] = zeros
        cnt_v[r, pl.ds(8, 16)] = zeros
    cp.wait()
    ones = jnp.ones((16,), jnp.float32)
    for i in range(_E // 16):
        s = edge_v[0, pl.ds(i * 16, 16)]
        d = edge_v[1, pl.ds(i * 16, 16)]
        plsc.addupdate_scatter(cnt_v, [d, s], ones)
    pltpu.sync_copy(cnt_v, out_hbm)


def _edge_hist(edge_index):
    mesh = plsc.VectorSubcoreMesh(core_axis_name="c", subcore_axis_name="s",
                                  num_cores=1, num_subcores=1)
    return pl.kernel(
        _edge_hist_body,
        out_type=jax.ShapeDtypeStruct((_N, _N), jnp.float32),
        mesh=mesh,
        compiler_params=pltpu.CompilerParams(
            needs_layout_passes=False,
            skip_device_barrier=True,
            disable_bounds_checks=True,
            disable_semaphore_checks=True,
        ),
        scratch_types=[
            pltpu.VMEM((2, _E), jnp.int32),
            pltpu.VMEM((_N, _N), jnp.float32),
            pltpu.SemaphoreType.DMA,
        ],
    )(edge_index)


def _dense_body(x_ref, c_ref, w1_ref, b1_ref, w2_ref, b2_ref, w3_ref, b3_ref,
                fc1_ref, fc1b_ref, fc2_ref, fc2b_ref, out_ref):
    f32 = jnp.float32

    def mm(a, b):
        return jnp.dot(a, b, preferred_element_type=f32,
                       precision=lax.Precision.HIGHEST)

    def mm_def(a, b):
        return jnp.dot(a, b, preferred_element_type=f32,
                       precision=lax.Precision.DEFAULT)

    C = (c_ref[0] + c_ref[1]).astype(f32)
    deg2 = mm(jnp.ones((1, _N), f32), C)
    safe = jnp.where(deg2 > 0.0, deg2, 1.0)
    dis2 = jnp.where(deg2 > 0.0, 1.0 / jnp.sqrt(safe), 0.0)

    r = lax.broadcasted_iota(jnp.int32, (_N, _N), 0)
    cc = lax.broadcasted_iota(jnp.int32, (_N, _N), 1)
    eye = r == cc
    D = jnp.where(eye, jnp.broadcast_to(dis2, (_N, _N)), 0.0)
    L = -mm(mm(D, C), D)
    degb = jnp.broadcast_to(deg2, (_N, _N))
    L = jnp.where(eye & (degb == 0.0), L - 1.0, L)

    def cheb(h, w_ref, b_ref, K):
        out = mm_def(h, w_ref[0])
        Tx0 = h
        Tx1 = mm(L, h)
        out = out + mm_def(Tx1, w_ref[1])
        for k in range(2, K):
            Tx2 = 2.0 * mm(L, Tx1) - Tx0
            out = out + mm_def(Tx2, w_ref[k])
            Tx0, Tx1 = Tx1, Tx2
        return out + b_ref[:].reshape(1, -1)

    def elu(v):
        return jnp.where(v > 0.0, v, jnp.exp(jnp.minimum(v, 0.0)) - 1.0)

    h = elu(cheb(x_ref[:], w1_ref, b1_ref, 3))
    h = elu(cheb(h, w2_ref, b2_ref, 3))
    h = elu(cheb(h, w3_ref, b3_ref, 5))

    acc = [fc1b_ref[:].reshape(1, -1),
           jnp.zeros((1, 128), f32), jnp.zeros((1, 128), f32),
           jnp.zeros((1, 128), f32)]
    for n in range(_N):
        acc[n % 4] = acc[n % 4] + mm_def(
            lax.slice(h, (n, 0), (n + 1, 32)), fc1_ref[n])
    y = (acc[0] + acc[1]) + (acc[2] + acc[3])
    z = mm_def(y, fc2_ref[:]) + fc2b_ref[:].reshape(1, 2)

    m = jnp.max(z, axis=1, keepdims=True)
    e = jnp.exp(z - m)
    lse = m + jnp.log(jnp.sum(e, axis=1, keepdims=True))
    out_ref[:] = z - lse


def _dense_call(x, cmat, W1, b1, W2, b2, W3, b3, fc1_wr, fc1_b, fc2_w, fc2_b):
    return pl.pallas_call(
        _dense_body,
        out_shape=jax.ShapeDtypeStruct((1, 2), jnp.float32),
    )(x, cmat, W1, b1, W2, b2, W3, b3, fc1_wr, fc1_b, fc2_w, fc2_b)


def kernel(x, edge_index, W1, b1, W2, b2, W3, b3, fc1_w, fc1_b, fc2_w, fc2_b):
    cmat = _edge_hist_scs(edge_index, jnp.zeros((2, _N, _N), jnp.int32))
    fc1_wr = fc1_w.reshape(_N, 32, 128)
    return _dense_call(x, cmat, W1, b1, W2, b2, W3, b3,
                       fc1_wr, fc1_b, fc2_w, fc2_b)


# speedup vs baseline: 1.0240x; 1.0240x over previous
import jax
import jax.numpy as jnp
from jax import lax
from jax.experimental import pallas as pl
from jax.experimental.pallas import tpu as pltpu
from jax.experimental.pallas import tpu_sc as plsc

_N = 24
_E = 128


def _edge_hist_scs_body(edge_hbm, zeros_hbm, out_hbm, edge_s, cnt_s,
                        sem1, sem2):
    cp1 = pltpu.async_copy(edge_hbm, edge_s, sem1)
    cp2 = pltpu.async_copy(zeros_hbm, cnt_s, sem2)
    cp1.wait()
    cp2.wait()

    def acc(e, _):
        s = edge_s[0, e]
        d = edge_s[1, e]
        cnt_s[d, s] = cnt_s[d, s] + 1
        return 0

    lax.fori_loop(0, _E, acc, 0)
    pltpu.sync_copy(cnt_s, out_hbm)


def _edge_hist_scs(edge_index, zeros):
    mesh = plsc.ScalarSubcoreMesh(axis_name="c", num_cores=1)
    return pl.kernel(
        _edge_hist_scs_body,
        out_type=jax.ShapeDtypeStruct((_N, _N), jnp.int32),
        mesh=mesh,
        compiler_params=pltpu.CompilerParams(
            needs_layout_passes=False,
            skip_device_barrier=True,
            disable_bounds_checks=True,
            disable_semaphore_checks=True,
        ),
        scratch_types=[
            pltpu.SMEM((2, _E), jnp.int32),
            pltpu.SMEM((_N, _N), jnp.int32),
            pltpu.SemaphoreType.DMA,
            pltpu.SemaphoreType.DMA,
        ],
    )(edge_index, zeros)


def _dense_body(x_ref, c_ref, w1_ref, b1_ref, w2_ref, b2_ref, w3_ref, b3_ref,
                fc1_ref, fc1b_ref, fc2_ref, fc2b_ref, out_ref):
    f32 = jnp.float32

    def mm(a, b):
        return jnp.dot(a, b, preferred_element_type=f32,
                       precision=lax.Precision.HIGHEST)

    def mm_def(a, b):
        return jnp.dot(a, b, preferred_element_type=f32,
                       precision=lax.Precision.DEFAULT)

    C = c_ref[:].astype(f32)
    deg2 = mm(jnp.ones((1, _N), f32), C)
    safe = jnp.where(deg2 > 0.0, deg2, 1.0)
    dis2 = jnp.where(deg2 > 0.0, 1.0 / jnp.sqrt(safe), 0.0)

    r = lax.broadcasted_iota(jnp.int32, (_N, _N), 0)
    cc = lax.broadcasted_iota(jnp.int32, (_N, _N), 1)
    eye = r == cc
    D = jnp.where(eye, jnp.broadcast_to(dis2, (_N, _N)), 0.0)
    L = -mm(mm(D, C), D)
    degb = jnp.broadcast_to(deg2, (_N, _N))
    L = jnp.where(eye & (degb == 0.0), L - 1.0, L)

    def cheb(h, w_ref, b_ref, K):
        out = mm_def(h, w_ref[0])
        Tx0 = h
        Tx1 = mm(L, h)
        out = out + mm_def(Tx1, w_ref[1])
        for k in range(2, K):
            Tx2 = 2.0 * mm(L, Tx1) - Tx0
            out = out + mm_def(Tx2, w_ref[k])
            Tx0, Tx1 = Tx1, Tx2
        return out + b_ref[:].reshape(1, -1)

    def elu(v):
        return jnp.where(v > 0.0, v, jnp.exp(jnp.minimum(v, 0.0)) - 1.0)

    h = elu(cheb(x_ref[:], w1_ref, b1_ref, 3))
    h = elu(cheb(h, w2_ref, b2_ref, 3))
    h = elu(cheb(h, w3_ref, b3_ref, 5))

    acc = [fc1b_ref[:].reshape(1, -1),
           jnp.zeros((1, 128), f32), jnp.zeros((1, 128), f32),
           jnp.zeros((1, 128), f32)]
    for n in range(_N):
        acc[n % 4] = acc[n % 4] + mm_def(
            lax.slice(h, (n, 0), (n + 1, 32)), fc1_ref[n])
    y = (acc[0] + acc[1]) + (acc[2] + acc[3])
    z = mm_def(y, fc2_ref[:]) + fc2b_ref[:].reshape(1, 2)

    m = jnp.max(z, axis=1, keepdims=True)
    e = jnp.exp(z - m)
    lse = m + jnp.log(jnp.sum(e, axis=1, keepdims=True))
    out_ref[:] = z - lse


def _dense_call(x, cmat, W1, b1, W2, b2, W3, b3, fc1_wr, fc1_b, fc2_w, fc2_b):
    return pl.pallas_call(
        _dense_body,
        out_shape=jax.ShapeDtypeStruct((1, 2), jnp.float32),
    )(x, cmat, W1, b1, W2, b2, W3, b3, fc1_wr, fc1_b, fc2_w, fc2_b)


def kernel(x, edge_index, W1, b1, W2, b2, W3, b3, fc1_w, fc1_b, fc2_w, fc2_b):
    cmat = _edge_hist_scs(edge_index, jnp.zeros((_N, _N), jnp.int32))
    fc1_wr = fc1_w.reshape(_N, 32, 128)
    return _dense_call(x, cmat, W1, b1, W2, b2, W3, b3,
                       fc1_wr, fc1_b, fc2_w, fc2_b)
